# baseline (speedup 1.0000x reference)
import functools

import jax
import jax.numpy as jnp
from jax import lax
from jax.experimental import pallas as pl
from jax.experimental.pallas import tpu as pltpu
from jax.experimental.pallas import tpu_sc as plsc

_B, _H, _W, _DIM = 16, 32, 32, 256


def _sc_body(row_hbm, col_hbm, img_hbm, plane_v, sem):
    c = lax.axis_index("c")
    s = lax.axis_index("s")
    h = s * 2 + c
    pltpu.sync_copy(col_hbm.at[pl.ds(0, _W)], plane_v.at[:, pl.ds(0, _DIM)])
    pltpu.sync_copy(row_hbm.at[h], plane_v.at[0, pl.ds(_DIM, _DIM)])
    for k in range(_DIM // 16):
        v = plane_v[0, pl.ds(_DIM + k * 16, 16)]
        for w in range(1, _W):
            plane_v[w, pl.ds(_DIM + k * 16, 16)] = v
    pltpu.async_copy(plane_v, img_hbm.at[h], sem).wait()


@functools.partial(
    pl.kernel,
    mesh=plsc.VectorSubcoreMesh(core_axis_name="c", subcore_axis_name="s"),
    out_type=jax.ShapeDtypeStruct((_H, _W, 2 * _DIM), jnp.float32),
    scratch_types=[
        pltpu.VMEM((_W, 2 * _DIM), jnp.float32),
        pltpu.SemaphoreType.DMA,
    ],
)
def _sc_lookup(row_hbm, col_hbm, img_hbm, plane_v, sem):
    _sc_body(row_hbm, col_hbm, img_hbm, plane_v, sem)


def _tc_tile_body(img_ref, out_ref):
    out_ref[0] = img_ref[...]


def kernel(inputs, row_embed, col_embed):
    b = inputs.shape[0]
    img = _sc_lookup(row_embed, col_embed)
    return pl.pallas_call(
        _tc_tile_body,
        grid=(b,),
        in_specs=[pl.BlockSpec((_H, _W, 2 * _DIM), lambda i: (0, 0, 0))],
        out_specs=pl.BlockSpec((1, _H, _W, 2 * _DIM), lambda i: (i, 0, 0, 0)),
        out_shape=jax.ShapeDtypeStruct((b, _H, _W, 2 * _DIM), jnp.float32),
    )(img)
